# baseline (speedup 1.0000x reference)
import functools

import jax
import jax.numpy as jnp
from jax import lax
from jax.experimental import pallas as pl
from jax.experimental.pallas import tpu as pltpu
from jax.experimental.pallas import tpu_sc as plsc

B, T, D = 4, 4096, 1024
NC, NS = 2, 16
NW = NC * NS
S_PER_W = T // NW
CH = 8
NCHUNK = S_PER_W // CH
NSLOT = 3
UNROLL = 32

_mesh = plsc.VectorSubcoreMesh(core_axis_name="c", subcore_axis_name="s")


@functools.partial(
    pl.kernel,
    mesh=_mesh,
    out_type=jax.ShapeDtypeStruct((B, T, D), jnp.float32),
    scratch_types=[
        pltpu.VMEM((B, CH, D), jnp.float32),
        pltpu.VMEM((B, CH, D), jnp.float32),
        pltpu.VMEM((B, CH, D), jnp.float32),
        pltpu.VMEM((CH, D), jnp.float32),
        pltpu.VMEM((CH, D), jnp.float32),
        pltpu.VMEM((CH, D), jnp.float32),
        pltpu.SemaphoreType.DMA,
        pltpu.SemaphoreType.DMA,
    ],
)
def _sc_add(x_h, emb_h, out_h, xb0, xb1, xb2, eb0, eb1, eb2, lsem, ssem):
    wid = lax.axis_index("s") * NC + lax.axis_index("c")
    base = wid * S_PER_W
    xbufs = (xb0, xb1, xb2)
    ebufs = (eb0, eb1, eb2)

    def fire_loads(c, slot):
        s0 = base + c * CH
        pltpu.async_copy(emb_h.at[pl.ds(s0, CH), :], ebufs[slot], lsem)
        pltpu.async_copy(x_h.at[:, pl.ds(s0, CH), :], xbufs[slot], lsem)

    def drain_loads(c, slot):
        s0 = base + c * CH
        pltpu.make_async_copy(
            emb_h.at[pl.ds(s0, CH), :], ebufs[slot], lsem
        ).wait()
        pltpu.make_async_copy(
            x_h.at[:, pl.ds(s0, CH), :], xbufs[slot], lsem
        ).wait()

    def fire_stores(c, slot):
        s0 = base + c * CH
        pltpu.async_copy(xbufs[slot], out_h.at[:, pl.ds(s0, CH), :], ssem)

    def drain_stores(c, slot):
        s0 = base + c * CH
        pltpu.make_async_copy(
            xbufs[slot], out_h.at[:, pl.ds(s0, CH), :], ssem
        ).wait()

    def compute(slot):
        xbuf, ebuf = xbufs[slot], ebufs[slot]

        def row_body(r, carry):
            def add_body(j, carry2):
                b0 = j * (16 * UNROLL)
                for u in range(UNROLL):
                    o = b0 + u * 16
                    e = ebuf[r, pl.ds(o, 16)]
                    for b in range(B):
                        plsc.addupdate(xbuf.at[b, r, pl.ds(o, 16)], e)
                return carry2

            return lax.fori_loop(0, D // (16 * UNROLL), add_body, carry)

        lax.fori_loop(0, CH, row_body, 0)

    fire_loads(0, 0)
    for c in range(NCHUNK):
        slot = c % NSLOT
        if c >= 2:
            drain_stores(c - 2, (c - 2) % NSLOT)
        if c + 1 < NCHUNK:
            fire_loads(c + 1, (c + 1) % NSLOT)
        drain_loads(c, slot)
        compute(slot)
        fire_stores(c, slot)
    drain_stores(NCHUNK - 2, (NCHUNK - 2) % NSLOT)
    drain_stores(NCHUNK - 1, (NCHUNK - 1) % NSLOT)


def kernel(x, emb):
    return _sc_add(x, emb)


# speedup vs baseline: 1.0864x; 1.0864x over previous
import functools

import jax
import jax.numpy as jnp
from jax import lax
from jax.experimental import pallas as pl
from jax.experimental.pallas import tpu as pltpu
from jax.experimental.pallas import tpu_sc as plsc

B, T, D = 4, 4096, 1024
NC, NS = 2, 16
NW = NC * NS
S_PER_W = T // NW
CH = 8
NCHUNK = S_PER_W // CH
NSLOT = 3
UNROLL = 8

_mesh = plsc.VectorSubcoreMesh(core_axis_name="c", subcore_axis_name="s")


@functools.partial(
    pl.kernel,
    mesh=_mesh,
    out_type=jax.ShapeDtypeStruct((B, T, D), jnp.float32),
    scratch_types=[
        pltpu.VMEM((B, CH, D), jnp.float32),
        pltpu.VMEM((B, CH, D), jnp.float32),
        pltpu.VMEM((B, CH, D), jnp.float32),
        pltpu.VMEM((CH, D), jnp.float32),
        pltpu.VMEM((CH, D), jnp.float32),
        pltpu.VMEM((CH, D), jnp.float32),
        pltpu.SemaphoreType.DMA,
        pltpu.SemaphoreType.DMA,
    ],
)
def _sc_add(x_h, emb_h, out_h, xb0, xb1, xb2, eb0, eb1, eb2, lsem, ssem):
    wid = lax.axis_index("s") * NC + lax.axis_index("c")
    base = wid * S_PER_W
    xbufs = (xb0, xb1, xb2)
    ebufs = (eb0, eb1, eb2)

    def fire_loads(c, slot):
        s0 = base + c * CH
        pltpu.async_copy(emb_h.at[pl.ds(s0, CH), :], ebufs[slot], lsem)
        pltpu.async_copy(x_h.at[:, pl.ds(s0, CH), :], xbufs[slot], lsem)

    def drain_loads(c, slot):
        s0 = base + c * CH
        pltpu.make_async_copy(
            emb_h.at[pl.ds(s0, CH), :], ebufs[slot], lsem
        ).wait()
        pltpu.make_async_copy(
            x_h.at[:, pl.ds(s0, CH), :], xbufs[slot], lsem
        ).wait()

    def fire_stores(c, slot):
        s0 = base + c * CH
        pltpu.async_copy(xbufs[slot], out_h.at[:, pl.ds(s0, CH), :], ssem)

    def drain_stores(c, slot):
        s0 = base + c * CH
        pltpu.make_async_copy(
            xbufs[slot], out_h.at[:, pl.ds(s0, CH), :], ssem
        ).wait()

    def compute(slot):
        xbuf, ebuf = xbufs[slot], ebufs[slot]

        def row_body(r, carry):
            def add_body(j, carry2):
                b0 = j * (16 * UNROLL)
                es = [
                    ebuf[r, pl.ds(b0 + u * 16, 16)] for u in range(UNROLL)
                ]
                for b in range(B):
                    for u in range(UNROLL):
                        o = b0 + u * 16
                        plsc.addupdate(xbuf.at[b, r, pl.ds(o, 16)], es[u])
                return carry2

            return lax.fori_loop(0, D // (16 * UNROLL), add_body, carry)

        lax.fori_loop(0, CH, row_body, 0)

    fire_loads(0, 0)
    for c in range(NCHUNK):
        slot = c % NSLOT
        if c >= 2:
            drain_stores(c - 2, (c - 2) % NSLOT)
        if c + 1 < NCHUNK:
            fire_loads(c + 1, (c + 1) % NSLOT)
        drain_loads(c, slot)
        compute(slot)
        fire_stores(c, slot)
    drain_stores(NCHUNK - 2, (NCHUNK - 2) % NSLOT)
    drain_stores(NCHUNK - 1, (NCHUNK - 1) % NSLOT)


def kernel(x, emb):
    return _sc_add(x, emb)
